# baseline (speedup 1.0000x reference)
import functools

import jax
import jax.numpy as jnp
from jax.experimental import pallas as pl
from jax.experimental.pallas import tpu as pltpu

_BLOCK = 512


def _router_block(x_ref, w1_ref, b1_ref, w2_ref, b2_ref, wts_ref, logits_ref):
    xb = x_ref[...].astype(jnp.bfloat16)
    w1 = w1_ref[...].astype(jnp.bfloat16)
    h = jnp.dot(xb, w1, preferred_element_type=jnp.float32)
    h = jnp.maximum(h + b1_ref[...], 0.0)
    logits = jnp.dot(h, w2_ref[...], preferred_element_type=jnp.float32)
    logits = logits + b2_ref[...]
    m = jnp.max(logits, axis=1, keepdims=True)
    e = jnp.exp(logits - m)
    wts_ref[...] = e / jnp.sum(e, axis=1, keepdims=True)
    logits_ref[...] = logits


@functools.partial(jax.jit, static_argnames=())
def kernel(x, W1, b1, W2, b2):
    n_tokens, feat_dim = x.shape
    hidden = W1.shape[1]
    n_experts = W2.shape[1]
    block = min(_BLOCK, n_tokens)
    grid = (n_tokens // block,)

    b1r = b1.reshape(1, hidden)
    b2r = b2.reshape(1, n_experts)

    wts, logits = pl.pallas_call(
        _router_block,
        grid=grid,
        in_specs=[
            pl.BlockSpec((block, feat_dim), lambda i: (i, 0)),
            pl.BlockSpec((feat_dim, hidden), lambda i: (0, 0)),
            pl.BlockSpec((1, hidden), lambda i: (0, 0)),
            pl.BlockSpec((hidden, n_experts), lambda i: (0, 0)),
            pl.BlockSpec((1, n_experts), lambda i: (0, 0)),
        ],
        out_specs=[
            pl.BlockSpec((block, n_experts), lambda i: (i, 0)),
            pl.BlockSpec((block, n_experts), lambda i: (i, 0)),
        ],
        out_shape=[
            jax.ShapeDtypeStruct((n_tokens, n_experts), jnp.float32),
            jax.ShapeDtypeStruct((n_tokens, n_experts), jnp.float32),
        ],
        compiler_params=pltpu.CompilerParams(
            dimension_semantics=("parallel",),
        ),
    )(x, W1, b1r, W2, b2r)
    return (wts, logits)


# speedup vs baseline: 1.4565x; 1.4565x over previous
import functools

import jax
import jax.numpy as jnp
from jax.experimental import pallas as pl
from jax.experimental.pallas import tpu as pltpu

_BLOCK = 4096
_SPLIT = 4


def _router_block(*refs):
    k = _SPLIT
    x_refs = refs[:k]
    w1_ref, b1_ref, w2_ref, b2_ref, wts_ref, logits_ref = refs[k:]
    w1 = w1_ref[...].astype(jnp.bfloat16)
    sub = x_refs[0].shape[0]
    for j in range(k):
        xb = x_refs[j][...].astype(jnp.bfloat16)
        h = jnp.dot(xb, w1, preferred_element_type=jnp.float32)
        h = jnp.maximum(h + b1_ref[...], 0.0)
        logits = jnp.dot(h, w2_ref[...], preferred_element_type=jnp.float32)
        logits = logits + b2_ref[...]
        m = jnp.max(logits, axis=1, keepdims=True)
        e = jnp.exp(logits - m)
        rows = pl.ds(j * sub, sub)
        wts_ref[rows, :] = e / jnp.sum(e, axis=1, keepdims=True)
        logits_ref[rows, :] = logits


@functools.partial(jax.jit, static_argnames=())
def kernel(x, W1, b1, W2, b2):
    n_tokens, feat_dim = x.shape
    hidden = W1.shape[1]
    n_experts = W2.shape[1]
    block = min(_BLOCK, n_tokens)
    k = _SPLIT
    sub = block // k
    grid = (n_tokens // block,)

    b1r = b1.reshape(1, hidden)
    b2r = b2.reshape(1, n_experts)

    x_specs = [
        pl.BlockSpec((sub, feat_dim), functools.partial(lambda j, i: (i * k + j, 0), j))
        for j in range(k)
    ]

    wts, logits = pl.pallas_call(
        _router_block,
        grid=grid,
        in_specs=x_specs + [
            pl.BlockSpec((feat_dim, hidden), lambda i: (0, 0)),
            pl.BlockSpec((1, hidden), lambda i: (0, 0)),
            pl.BlockSpec((hidden, n_experts), lambda i: (0, 0)),
            pl.BlockSpec((1, n_experts), lambda i: (0, 0)),
        ],
        out_specs=[
            pl.BlockSpec((block, n_experts), lambda i: (i, 0)),
            pl.BlockSpec((block, n_experts), lambda i: (i, 0)),
        ],
        out_shape=[
            jax.ShapeDtypeStruct((n_tokens, n_experts), jnp.float32),
            jax.ShapeDtypeStruct((n_tokens, n_experts), jnp.float32),
        ],
        compiler_params=pltpu.CompilerParams(
            dimension_semantics=("arbitrary",),
        ),
    )(*([x] * k), W1, b1r, W2, b2r)
    return (wts, logits)


# speedup vs baseline: 1.6215x; 1.1133x over previous
import functools

import jax
import jax.numpy as jnp
from jax.experimental import pallas as pl
from jax.experimental.pallas import tpu as pltpu

_ROWS = 1024
_NBUF = 6


def _router_block(x_hbm, w1_ref, b1_ref, w2_ref, b2_ref,
                  wts_ref, logits_ref, buf, sems):
    i = pl.program_id(0)
    n_chunks = pl.num_programs(0)

    def start(chunk, slot):
        pltpu.make_async_copy(
            x_hbm.at[pl.ds(chunk * _ROWS, _ROWS), :],
            buf.at[slot],
            sems.at[slot],
        ).start()

    @pl.when(i == 0)
    def _prefetch():
        for j in range(min(_NBUF - 1, n_chunks)):
            start(j, j)

    nxt = i + _NBUF - 1

    @pl.when((i == 0) | (nxt < n_chunks))
    def _start_next():
        c = jnp.minimum(nxt, n_chunks - 1)
        start(c, c % _NBUF)

    slot = i % _NBUF
    pltpu.make_async_copy(
        x_hbm.at[pl.ds(i * _ROWS, _ROWS), :],
        buf.at[slot],
        sems.at[slot],
    ).wait()

    xb = buf[slot].astype(jnp.bfloat16)
    w1 = w1_ref[...].astype(jnp.bfloat16)
    h = jnp.dot(xb, w1, preferred_element_type=jnp.float32)
    h = jnp.maximum(h + b1_ref[...], 0.0)
    logits = jnp.dot(h, w2_ref[...], preferred_element_type=jnp.float32)
    logits = logits + b2_ref[...]
    m = jnp.max(logits, axis=1, keepdims=True)
    e = jnp.exp(logits - m)
    wts_ref[...] = e / jnp.sum(e, axis=1, keepdims=True)
    logits_ref[...] = logits


@functools.partial(jax.jit, static_argnames=())
def kernel(x, W1, b1, W2, b2):
    n_tokens, feat_dim = x.shape
    hidden = W1.shape[1]
    n_experts = W2.shape[1]
    n_chunks = n_tokens // _ROWS
    grid = (n_chunks,)

    b1r = b1.reshape(1, hidden)
    b2r = b2.reshape(1, n_experts)

    wts, logits = pl.pallas_call(
        _router_block,
        grid=grid,
        in_specs=[
            pl.BlockSpec(memory_space=pl.ANY),
            pl.BlockSpec((feat_dim, hidden), lambda i: (0, 0)),
            pl.BlockSpec((1, hidden), lambda i: (0, 0)),
            pl.BlockSpec((hidden, n_experts), lambda i: (0, 0)),
            pl.BlockSpec((1, n_experts), lambda i: (0, 0)),
        ],
        out_specs=[
            pl.BlockSpec((_ROWS, n_experts), lambda i: (i, 0)),
            pl.BlockSpec((_ROWS, n_experts), lambda i: (i, 0)),
        ],
        out_shape=[
            jax.ShapeDtypeStruct((n_tokens, n_experts), jnp.float32),
            jax.ShapeDtypeStruct((n_tokens, n_experts), jnp.float32),
        ],
        scratch_shapes=[
            pltpu.VMEM((_NBUF, _ROWS, feat_dim), jnp.float32),
            pltpu.SemaphoreType.DMA((_NBUF,)),
        ],
        compiler_params=pltpu.CompilerParams(
            dimension_semantics=("arbitrary",),
        ),
    )(x, W1, b1r, W2, b2r)
    return (wts, logits)
